# baseline (speedup 1.0000x reference)
import math
import functools

import numpy as np
import jax
import jax.numpy as jnp
from jax.experimental import pallas as pl
from jax.experimental.pallas import tpu as pltpu
from jax.experimental.pallas import tpu_sc as plsc

VOCAB = 100000
EMB = 64
MAX_LEN = 200
BATCH = 4096
SEQ = 200

GATHER = 128
WINDOW = 256
WPL = BATCH // WINDOW
NUM_ROWS = SEQ * BATCH
NUM_WINDOWS = NUM_ROWS // WINDOW


def _positional_encoding():
    position = np.arange(0, MAX_LEN, dtype=np.float32)[:, None]
    div_term = np.exp(
        np.arange(0, EMB, 2, dtype=np.float32) * np.float32(-(math.log(10000.0) / EMB))
    ).astype(np.float32)
    pe = np.zeros((MAX_LEN, EMB), dtype=np.float32)
    pe[:, 0::2] = np.sin(position * div_term, dtype=np.float32)
    pe[:, 1::2] = np.cos(position * div_term, dtype=np.float32)
    return pe


def _tc_transpose(x):
    def body(x_ref, o_ref):
        o_ref[...] = x_ref[...].T

    return pl.pallas_call(
        body,
        out_shape=jax.ShapeDtypeStruct((SEQ, BATCH), jnp.int32),
    )(x)


def _make_sc_kernel():
    mesh = plsc.VectorSubcoreMesh(core_axis_name="core", subcore_axis_name="subcore")

    @functools.partial(
        pl.kernel,
        out_type=jax.ShapeDtypeStruct((SEQ, BATCH, EMB), jnp.float32),
        mesh=mesh,
        compiler_params=pltpu.CompilerParams(use_tc_tiling_on_sc=True),
        scratch_types=[
            pltpu.VMEM((WINDOW, 128), jnp.float32),
            pltpu.SemaphoreType.DMA,
        ],
    )
    def sc_kernel(table_hbm, idx_hbm, pe_hbm, out_hbm, g_vmem, gsem):
        def body(i_vmem, pe_vmem, o_vmem):
            for j in range(WINDOW // GATHER):
                pltpu.async_copy(
                    table_hbm.at[i_vmem.at[pl.ds(j * GATHER, GATHER)]],
                    g_vmem.at[pl.ds(j * GATHER, GATHER), :],
                    gsem,
                )
            pltpu.make_async_copy(
                table_hbm.at[i_vmem.at[pl.ds(0, GATHER)]], g_vmem, gsem
            ).wait()
            pe_regs = [pe_vmem[0, pl.ds(16 * j, 16)] for j in range(EMB // 16)]

            @pl.loop(0, WINDOW, unroll=8)
            def _(r):
                for j in range(EMB // 16):
                    slc = pl.ds(16 * j, 16)
                    o_vmem[0, r, slc] = g_vmem[r, slc] + pe_regs[j]

        pltpu.emit_pipeline(
            body,
            grid=(NUM_WINDOWS,),
            in_specs=[
                pl.BlockSpec((WINDOW,), index_map=lambda i: (i,)),
                pl.BlockSpec((1, 128), index_map=lambda i: (i, 0)),
            ],
            out_specs=[
                pl.BlockSpec(
                    (1, WINDOW, EMB), index_map=lambda i: (i // WPL, i % WPL, 0)
                ),
            ],
            core_axis_name=("core", "subcore"),
            dimension_semantics=(pltpu.PARALLEL,),
        )(idx_hbm, pe_hbm, out_hbm)

    return sc_kernel


_SC_KERNEL = _make_sc_kernel()


_PE_WIN = np.tile(np.repeat(_positional_encoding(), BATCH // WINDOW, axis=0), (1, 2))


def kernel(input, table):
    idx_t = _tc_transpose(input.astype(jnp.int32)).reshape(NUM_ROWS)
    table_pad = jnp.pad(table, ((0, 0), (0, 128 - EMB)))
    return _SC_KERNEL(table_pad, idx_t, _PE_WIN)


# speedup vs baseline: 1.5699x; 1.5699x over previous
import math
import functools

import numpy as np
import jax
import jax.numpy as jnp
from jax.experimental import pallas as pl
from jax.experimental.pallas import tpu as pltpu
from jax.experimental.pallas import tpu_sc as plsc

VOCAB = 100000
EMB = 64
MAX_LEN = 200
BATCH = 4096
SEQ = 200

GATHER = 128
WINDOW = 512
NUM_ROWS = SEQ * BATCH
NUM_WINDOWS = NUM_ROWS // WINDOW


def _positional_encoding():
    position = np.arange(0, MAX_LEN, dtype=np.float32)[:, None]
    div_term = np.exp(
        np.arange(0, EMB, 2, dtype=np.float32) * np.float32(-(math.log(10000.0) / EMB))
    ).astype(np.float32)
    pe = np.zeros((MAX_LEN, EMB), dtype=np.float32)
    pe[:, 0::2] = np.sin(position * div_term, dtype=np.float32)
    pe[:, 1::2] = np.cos(position * div_term, dtype=np.float32)
    return pe


def _tc_transpose(x):
    def body(x_ref, o_ref):
        o_ref[...] = x_ref[...].T

    return pl.pallas_call(
        body,
        out_shape=jax.ShapeDtypeStruct((SEQ, BATCH), jnp.int32),
    )(x)


def _make_sc_kernel():
    mesh = plsc.VectorSubcoreMesh(core_axis_name="core", subcore_axis_name="subcore")

    @functools.partial(
        pl.kernel,
        out_type=jax.ShapeDtypeStruct((NUM_ROWS, EMB), jnp.float32),
        mesh=mesh,
        compiler_params=pltpu.CompilerParams(use_tc_tiling_on_sc=False),
        scratch_types=[pltpu.SemaphoreType.DMA] * (WINDOW // GATHER),
    )
    def sc_kernel(table_hbm, idx_hbm, pe_hbm, out_hbm, *sems):
        def body(i_vmem, pe_vmem, o_vmem):
            for j in range(WINDOW // GATHER):
                pltpu.async_copy(
                    table_hbm.at[i_vmem.at[0, pl.ds(j * GATHER, GATHER)]],
                    o_vmem.at[pl.ds(j * GATHER, GATHER), :],
                    sems[j],
                )
            pe_regs = [pe_vmem[0, pl.ds(16 * j, 16)] for j in range(EMB // 16)]

            for j in range(WINDOW // GATHER):
                pltpu.make_async_copy(
                    table_hbm.at[i_vmem.at[0, pl.ds(j * GATHER, GATHER)]],
                    o_vmem.at[pl.ds(j * GATHER, GATHER), :],
                    sems[j],
                ).wait()

                @pl.loop(j * GATHER, (j + 1) * GATHER, unroll=8)
                def _(r):
                    for k in range(EMB // 16):
                        plsc.addupdate(o_vmem.at[r, pl.ds(16 * k, 16)], pe_regs[k])

        pltpu.emit_pipeline(
            body,
            grid=(NUM_WINDOWS,),
            in_specs=[
                pl.BlockSpec((1, WINDOW), index_map=lambda i: (0, i)),
                pl.BlockSpec((1, EMB), index_map=lambda i: (i, 0)),
            ],
            out_specs=[
                pl.BlockSpec((WINDOW, EMB), index_map=lambda i: (i, 0)),
            ],
            core_axis_name=("core", "subcore"),
            dimension_semantics=(pltpu.PARALLEL,),
        )(idx_hbm, pe_hbm, out_hbm)

    return sc_kernel


_SC_KERNEL = _make_sc_kernel()


_PE_WIN = np.repeat(
    _positional_encoding(), BATCH // WINDOW, axis=0
)


def kernel(input, table):
    idx_t = _tc_transpose(input.astype(jnp.int32)).reshape(1, NUM_ROWS)
    out_flat = _SC_KERNEL(table, idx_t, _PE_WIN)
    return out_flat.reshape(SEQ, BATCH, EMB)
